# baseline (speedup 1.0000x reference)
import functools

import jax
import jax.numpy as jnp
from jax import lax
from jax.experimental import pallas as pl
from jax.experimental.pallas import tpu as pltpu
from jax.experimental.pallas import tpu_sc as plsc

K = 256
_NW = 32
_L = 16
_KC = 4


def _sc_segsum(x3, labs_flat):
    B, C, HW = x3.shape
    spt = HW // (_NW // B)
    ng = spt // _L
    nco = C // _KC

    mesh = plsc.VectorSubcoreMesh(core_axis_name="c", subcore_axis_name="s")

    @functools.partial(
        pl.kernel,
        out_type=(
            jax.ShapeDtypeStruct((_NW, K * C), jnp.float32),
            jax.ShapeDtypeStruct((_NW, K), jnp.float32),
        ),
        mesh=mesh,
        scratch_types=[
            pltpu.VMEM((spt,), jnp.int32),
            pltpu.VMEM((2, _KC, spt), jnp.float32),
            pltpu.VMEM((K * C,), jnp.float32),
            pltpu.VMEM((K,), jnp.float32),
            pltpu.SemaphoreType.DMA,
            pltpu.SemaphoreType.DMA,
        ],
        compiler_params=pltpu.CompilerParams(needs_layout_passes=False),
    )
    def segsum_kernel(x_hbm, labs_hbm, sums_hbm, cnt_hbm,
                      lab_v, buf, hist_v, cnt_v, sem0, sem1):
        wid = lax.axis_index("s") * 2 + lax.axis_index("c")
        img = wid // (_NW // B)
        sl = wid % (_NW // B)
        pbase = sl * spt
        sems = (sem0, sem1)

        def mk(i, t):
            return pltpu.make_async_copy(
                x_hbm.at[img, pl.ds(i * _KC, _KC), pl.ds(pbase, spt)],
                buf.at[t], sems[t])

        mk(0, 0).start()
        mk(1, 1).start()

        pltpu.sync_copy(labs_hbm.at[pl.ds(wid * spt, spt)], lab_v)

        zv = jnp.zeros((_L,), jnp.float32)

        def zbody(z, carry):
            base = z * (_L * 8)
            for u in range(8):
                hist_v[pl.ds(base + u * _L, _L)] = zv
            return carry

        lax.fori_loop(0, (K * C) // (_L * 8), zbody, 0)
        for u in range(K // _L):
            cnt_v[pl.ds(u * _L, _L)] = zv

        ones = jnp.ones((_L,), jnp.float32)

        def cbody(g, carry):
            lv = lab_v[pl.ds(g * _L, _L)]
            plsc.addupdate_scatter(cnt_v, [lv], ones)
            return carry

        lax.fori_loop(0, ng, cbody, 0)

        def body(i2, carry):
            for t in range(2):
                i = i2 * 2 + t
                mk(i, t).wait()

                def gbody(g, carry2):
                    lv = lab_v[pl.ds(g * _L, _L)]
                    base = lv * C + (i * _KC)
                    for ci in range(_KC):
                        xv = buf[t, ci, pl.ds(g * _L, _L)]
                        plsc.addupdate_scatter(hist_v, [base + ci], xv)
                    return carry2

                lax.fori_loop(0, ng, gbody, 0)

                @pl.when(i + 2 < nco)
                def _():
                    mk(i + 2, t).start()
            return carry

        lax.fori_loop(0, nco // 2, body, 0)

        pltpu.sync_copy(hist_v, sums_hbm.at[wid])
        pltpu.sync_copy(cnt_v, cnt_hbm.at[wid])

    return segsum_kernel(x3, labs_flat)


def _finalize_body(sp_ref, cp_ref, out_ref):
    s = jnp.sum(sp_ref[0], axis=0)
    ones_col = jnp.ones((cp_ref.shape[1], 1), jnp.float32)
    counts = lax.dot_general(
        cp_ref[0], ones_col, (((0,), (0,)), ((), ())),
        preferred_element_type=jnp.float32)
    out_ref[0] = s / jnp.maximum(counts, 1.0)


def kernel(x, label_maps):
    B, C, H, W = x.shape
    HW = H * W
    wpi = _NW // B

    x3 = x.reshape(B, C, HW)

    sums_p, cnt_p = _sc_segsum(x3, label_maps.reshape(-1))
    sums4 = sums_p.reshape(B, wpi, K, C)
    cnt3 = cnt_p.reshape(B, wpi, K)

    out = pl.pallas_call(
        _finalize_body,
        grid=(B,),
        in_specs=[
            pl.BlockSpec((1, wpi, K, C), lambda b: (b, 0, 0, 0)),
            pl.BlockSpec((1, wpi, K), lambda b: (b, 0, 0)),
        ],
        out_specs=pl.BlockSpec((1, K, C), lambda b: (b, 0, 0)),
        out_shape=jax.ShapeDtypeStruct((B, K, C), jnp.float32),
        compiler_params=pltpu.CompilerParams(
            dimension_semantics=("arbitrary",)),
    )(sums4, cnt3)
    return out


# speedup vs baseline: 1.2343x; 1.2343x over previous
import functools

import jax
import jax.numpy as jnp
from jax import lax
from jax.experimental import pallas as pl
from jax.experimental.pallas import tpu as pltpu
from jax.experimental.pallas import tpu_sc as plsc

K = 256
_NW = 32
_L = 16
_KC = 4


def _sc_segsum(x3, labs_flat):
    B, C, HW = x3.shape
    spt = HW // (_NW // B)
    ng = spt // _L
    nco = C // _KC

    mesh = plsc.VectorSubcoreMesh(core_axis_name="c", subcore_axis_name="s")

    @functools.partial(
        pl.kernel,
        out_type=(
            jax.ShapeDtypeStruct((_NW, K * C), jnp.float32),
            jax.ShapeDtypeStruct((_NW, K), jnp.float32),
        ),
        mesh=mesh,
        scratch_types=[
            pltpu.VMEM((spt,), jnp.int32),
            pltpu.VMEM((2, _KC, spt), jnp.float32),
            pltpu.VMEM((K * C,), jnp.float32),
            pltpu.VMEM((K,), jnp.float32),
            pltpu.SemaphoreType.DMA,
            pltpu.SemaphoreType.DMA,
        ],
        compiler_params=pltpu.CompilerParams(needs_layout_passes=False),
    )
    def segsum_kernel(x_hbm, labs_hbm, sums_hbm, cnt_hbm,
                      lab_v, buf, hist_v, cnt_v, sem0, sem1):
        wid = lax.axis_index("s") * 2 + lax.axis_index("c")
        img = wid // (_NW // B)
        sl = wid % (_NW // B)
        pbase = sl * spt
        sems = (sem0, sem1)

        def mk(i, t):
            return pltpu.make_async_copy(
                x_hbm.at[img, pl.ds(i * _KC, _KC), pl.ds(pbase, spt)],
                buf.at[t], sems[t])

        mk(0, 0).start()
        mk(1, 1).start()

        pltpu.sync_copy(labs_hbm.at[pl.ds(wid * spt, spt)], lab_v)

        zv = jnp.zeros((_L,), jnp.float32)

        @plsc.parallel_loop(0, (K * C) // _L, 1, unroll=8)
        def _zero(z):
            hist_v[pl.ds(z * _L, _L)] = zv

        for u in range(K // _L):
            cnt_v[pl.ds(u * _L, _L)] = zv

        ones = jnp.ones((_L,), jnp.float32)

        @plsc.parallel_loop(0, ng, 1, unroll=8)
        def _counts(g):
            lv = lab_v[pl.ds(g * _L, _L)]
            plsc.addupdate_scatter(cnt_v, [lv], ones)

        def body(i2, carry):
            for t in range(2):
                i = i2 * 2 + t
                mk(i, t).wait()

                @plsc.parallel_loop(0, ng, 1, unroll=8)
                def _gather(g):
                    lv = lab_v[pl.ds(g * _L, _L)]
                    base = lv * C + (i * _KC)
                    for ci in range(_KC):
                        xv = buf[t, ci, pl.ds(g * _L, _L)]
                        plsc.addupdate_scatter(hist_v, [base + ci], xv)

                @pl.when(i + 2 < nco)
                def _():
                    mk(i + 2, t).start()
            return carry

        lax.fori_loop(0, nco // 2, body, 0)

        pltpu.sync_copy(hist_v, sums_hbm.at[wid])
        pltpu.sync_copy(cnt_v, cnt_hbm.at[wid])

    return segsum_kernel(x3, labs_flat)


def _finalize_body(sp_ref, cp_ref, out_ref):
    s = jnp.sum(sp_ref[0], axis=0)
    ones_col = jnp.ones((cp_ref.shape[1], 1), jnp.float32)
    counts = lax.dot_general(
        cp_ref[0], ones_col, (((0,), (0,)), ((), ())),
        preferred_element_type=jnp.float32)
    out_ref[0] = s / jnp.maximum(counts, 1.0)


def kernel(x, label_maps):
    B, C, H, W = x.shape
    HW = H * W
    wpi = _NW // B

    x3 = x.reshape(B, C, HW)

    sums_p, cnt_p = _sc_segsum(x3, label_maps.reshape(-1))
    sums4 = sums_p.reshape(B, wpi, K, C)
    cnt3 = cnt_p.reshape(B, wpi, K)

    out = pl.pallas_call(
        _finalize_body,
        grid=(B,),
        in_specs=[
            pl.BlockSpec((1, wpi, K, C), lambda b: (b, 0, 0, 0)),
            pl.BlockSpec((1, wpi, K), lambda b: (b, 0, 0)),
        ],
        out_specs=pl.BlockSpec((1, K, C), lambda b: (b, 0, 0)),
        out_shape=jax.ShapeDtypeStruct((B, K, C), jnp.float32),
        compiler_params=pltpu.CompilerParams(
            dimension_semantics=("arbitrary",)),
    )(sums4, cnt3)
    return out


# speedup vs baseline: 4.7678x; 3.8629x over previous
import functools

import jax
import jax.numpy as jnp
from jax import lax
from jax.experimental import pallas as pl
from jax.experimental.pallas import tpu as pltpu
from jax.experimental.pallas import tpu_sc as plsc

K = 256
_NW = 32
_L = 16


def _sc_counts(labs_flat):
    n = labs_flat.shape[0]
    lpw = n // _NW

    mesh = plsc.VectorSubcoreMesh(core_axis_name="c", subcore_axis_name="s")

    @functools.partial(
        pl.kernel,
        out_type=jax.ShapeDtypeStruct((_NW, K), jnp.float32),
        mesh=mesh,
        scratch_types=[
            pltpu.VMEM((lpw,), jnp.int32),
            pltpu.VMEM((K,), jnp.float32),
        ],
        compiler_params=pltpu.CompilerParams(needs_layout_passes=False),
    )
    def counts_kernel(labs_hbm, out_hbm, lab_v, hist_v):
        wid = lax.axis_index("s") * 2 + lax.axis_index("c")
        pltpu.sync_copy(labs_hbm.at[pl.ds(wid * lpw, lpw)], lab_v)
        zeros = jnp.zeros((_L,), jnp.float32)
        for i in range(K // _L):
            hist_v[pl.ds(i * _L, _L)] = zeros
        ones = jnp.ones((_L,), jnp.float32)

        @plsc.parallel_loop(0, lpw // _L, 1, unroll=8)
        def _hist(g):
            idx = lab_v[pl.ds(g * _L, _L)]
            plsc.addupdate_scatter(hist_v, [idx], ones)

        pltpu.sync_copy(hist_v, out_hbm.at[wid])

    return counts_kernel(labs_flat)


def _pool_body(nj, x_ref, lab_ref, cp_ref, out_ref):
    j = pl.program_id(1)

    labs = lab_ref[0]
    kiota = lax.broadcasted_iota(jnp.int32, (K, labs.shape[-1]), 0)
    onehot = (labs == kiota).astype(jnp.bfloat16)
    xb = x_ref[0].astype(jnp.bfloat16)

    psum = lax.dot_general(
        onehot, xb, (((1,), (1,)), ((), ())),
        preferred_element_type=jnp.float32)

    @pl.when(j == 0)
    def _init():
        out_ref[0] = psum

    @pl.when(j > 0)
    def _acc():
        out_ref[0] += psum

    @pl.when(j == nj - 1)
    def _finish():
        ones_col = jnp.ones((cp_ref.shape[1], 1), jnp.float32)
        counts = lax.dot_general(
            cp_ref[0], ones_col, (((0,), (0,)), ((), ())),
            preferred_element_type=jnp.float32)
        out_ref[0] = out_ref[0] / jnp.maximum(counts, 1.0)


def kernel(x, label_maps):
    B, C, H, W = x.shape
    HW = H * W
    chunk = 7168 if HW % 7168 == 0 else HW
    nj = HW // chunk
    wpi = _NW // B

    x3 = x.reshape(B, C, HW)
    labs = label_maps.reshape(B * nj, 1, chunk)

    partials = _sc_counts(label_maps.reshape(-1))
    cp = partials.reshape(B, wpi, K)

    out = pl.pallas_call(
        functools.partial(_pool_body, nj),
        grid=(B, nj),
        in_specs=[
            pl.BlockSpec((1, C, chunk), lambda b, j: (b, 0, j)),
            pl.BlockSpec((1, 1, chunk), lambda b, j: (b * nj + j, 0, 0)),
            pl.BlockSpec((1, wpi, K), lambda b, j: (b, 0, 0)),
        ],
        out_specs=pl.BlockSpec((1, K, C), lambda b, j: (b, 0, 0)),
        out_shape=jax.ShapeDtypeStruct((B, K, C), jnp.float32),
        compiler_params=pltpu.CompilerParams(
            dimension_semantics=("parallel", "arbitrary")),
    )(x3, labs, cp)
    return out


# speedup vs baseline: 4.9006x; 1.0278x over previous
import functools

import jax
import jax.numpy as jnp
from jax import lax
from jax.experimental import pallas as pl
from jax.experimental.pallas import tpu as pltpu
from jax.experimental.pallas import tpu_sc as plsc

K = 256
_NW = 32
_L = 16


def _sc_counts(labs_flat):
    n = labs_flat.shape[0]
    lpw = n // _NW

    mesh = plsc.VectorSubcoreMesh(core_axis_name="c", subcore_axis_name="s")

    @functools.partial(
        pl.kernel,
        out_type=jax.ShapeDtypeStruct((_NW, K), jnp.float32),
        mesh=mesh,
        scratch_types=[
            pltpu.VMEM((lpw,), jnp.int32),
            pltpu.VMEM((K,), jnp.float32),
        ],
        compiler_params=pltpu.CompilerParams(needs_layout_passes=False),
    )
    def counts_kernel(labs_hbm, out_hbm, lab_v, hist_v):
        wid = lax.axis_index("s") * 2 + lax.axis_index("c")
        pltpu.sync_copy(labs_hbm.at[pl.ds(wid * lpw, lpw)], lab_v)
        zeros = jnp.zeros((_L,), jnp.float32)
        for i in range(K // _L):
            hist_v[pl.ds(i * _L, _L)] = zeros
        ones = jnp.ones((_L,), jnp.float32)

        @plsc.parallel_loop(0, lpw // _L, 1, unroll=8)
        def _hist(g):
            idx = lab_v[pl.ds(g * _L, _L)]
            plsc.addupdate_scatter(hist_v, [idx], ones)

        pltpu.sync_copy(hist_v, out_hbm.at[wid])

    return counts_kernel(labs_flat)


def _pool_body(nj, x_ref, lab_ref, cp_ref, out_ref):
    j = pl.program_id(1)

    labs = lab_ref[0]
    kiota = lax.broadcasted_iota(jnp.int32, (K, labs.shape[-1]), 0)
    onehot = (labs == kiota).astype(jnp.bfloat16)
    xb = x_ref[0].astype(jnp.bfloat16)

    psum = lax.dot_general(
        onehot, xb, (((1,), (1,)), ((), ())),
        preferred_element_type=jnp.float32)

    @pl.when(j == 0)
    def _init():
        out_ref[0] = psum

    @pl.when(j > 0)
    def _acc():
        out_ref[0] += psum

    @pl.when(j == nj - 1)
    def _finish():
        ones_col = jnp.ones((cp_ref.shape[1], 1), jnp.float32)
        counts = lax.dot_general(
            cp_ref[0], ones_col, (((0,), (0,)), ((), ())),
            preferred_element_type=jnp.float32)
        out_ref[0] = out_ref[0] / jnp.maximum(counts, 1.0)


def kernel(x, label_maps):
    B, C, H, W = x.shape
    HW = H * W
    chunk = 12544 if HW % 12544 == 0 else HW
    nj = HW // chunk
    wpi = _NW // B

    x3 = x.reshape(B, C, HW)
    labs = label_maps.reshape(B * nj, 1, chunk)

    partials = _sc_counts(label_maps.reshape(-1))
    cp = partials.reshape(B, wpi, K)

    out = pl.pallas_call(
        functools.partial(_pool_body, nj),
        grid=(B, nj),
        in_specs=[
            pl.BlockSpec((1, C, chunk), lambda b, j: (b, 0, j)),
            pl.BlockSpec((1, 1, chunk), lambda b, j: (b * nj + j, 0, 0)),
            pl.BlockSpec((1, wpi, K), lambda b, j: (b, 0, 0)),
        ],
        out_specs=pl.BlockSpec((1, K, C), lambda b, j: (b, 0, 0)),
        out_shape=jax.ShapeDtypeStruct((B, K, C), jnp.float32),
        compiler_params=pltpu.CompilerParams(
            dimension_semantics=("parallel", "arbitrary")),
    )(x3, labs, cp)
    return out


# speedup vs baseline: 4.9054x; 1.0010x over previous
import functools

import jax
import jax.numpy as jnp
from jax import lax
from jax.experimental import pallas as pl
from jax.experimental.pallas import tpu as pltpu
from jax.experimental.pallas import tpu_sc as plsc

K = 256
_NW = 32
_L = 16


def _sc_counts(labs_flat):
    n = labs_flat.shape[0]
    lpw = n // _NW

    mesh = plsc.VectorSubcoreMesh(core_axis_name="c", subcore_axis_name="s")

    @functools.partial(
        pl.kernel,
        out_type=jax.ShapeDtypeStruct((_NW, K), jnp.float32),
        mesh=mesh,
        scratch_types=[
            pltpu.VMEM((lpw,), jnp.int32),
            pltpu.VMEM((K,), jnp.float32),
        ],
        compiler_params=pltpu.CompilerParams(needs_layout_passes=False),
    )
    def counts_kernel(labs_hbm, out_hbm, lab_v, hist_v):
        wid = lax.axis_index("s") * 2 + lax.axis_index("c")
        pltpu.sync_copy(labs_hbm.at[pl.ds(wid * lpw, lpw)], lab_v)
        zeros = jnp.zeros((_L,), jnp.float32)
        for i in range(K // _L):
            hist_v[pl.ds(i * _L, _L)] = zeros
        ones = jnp.ones((_L,), jnp.float32)

        @plsc.parallel_loop(0, lpw // _L, 1, unroll=8)
        def _hist(g):
            idx = lab_v[pl.ds(g * _L, _L)]
            plsc.addupdate_scatter(hist_v, [idx], ones)

        pltpu.sync_copy(hist_v, out_hbm.at[wid])

    return counts_kernel(labs_flat)


def _pool_body(nj, x_ref, lab_ref, cp_ref, out_ref):
    j = pl.program_id(1)

    labs = lab_ref[0]
    kiota = lax.broadcasted_iota(jnp.int32, (K, labs.shape[-1]), 0)
    onehot = (labs == kiota).astype(jnp.bfloat16)
    xb = x_ref[0].astype(jnp.bfloat16)

    psum = lax.dot_general(
        onehot, xb, (((1,), (1,)), ((), ())),
        preferred_element_type=jnp.float32)

    @pl.when(j == 0)
    def _init():
        out_ref[0] = psum

    @pl.when(j > 0)
    def _acc():
        out_ref[0] += psum

    @pl.when(j == nj - 1)
    def _finish():
        ones_col = jnp.ones((cp_ref.shape[1], 1), jnp.float32)
        counts = lax.dot_general(
            cp_ref[0], ones_col, (((0,), (0,)), ((), ())),
            preferred_element_type=jnp.float32)
        out_ref[0] = out_ref[0] / jnp.maximum(counts, 1.0)


def kernel(x, label_maps):
    B, C, H, W = x.shape
    HW = H * W
    chunk = 25088 if HW % 25088 == 0 else HW
    nj = HW // chunk
    wpi = _NW // B

    x3 = x.reshape(B, C, HW)
    labs = label_maps.reshape(B * nj, 1, chunk)

    partials = _sc_counts(label_maps.reshape(-1))
    cp = partials.reshape(B, wpi, K)

    out = pl.pallas_call(
        functools.partial(_pool_body, nj),
        grid=(B, nj),
        in_specs=[
            pl.BlockSpec((1, C, chunk), lambda b, j: (b, 0, j)),
            pl.BlockSpec((1, 1, chunk), lambda b, j: (b * nj + j, 0, 0)),
            pl.BlockSpec((1, wpi, K), lambda b, j: (b, 0, 0)),
        ],
        out_specs=pl.BlockSpec((1, K, C), lambda b, j: (b, 0, 0)),
        out_shape=jax.ShapeDtypeStruct((B, K, C), jnp.float32),
        compiler_params=pltpu.CompilerParams(
            dimension_semantics=("parallel", "arbitrary"),
            vmem_limit_bytes=60000 * 1024),
    )(x3, labs, cp)
    return out
